# Initial kernel scaffold; baseline (speedup 1.0000x reference)
import jax
import jax.numpy as jnp
from jax.experimental import pallas as pl


def kernel(node_feats, edge_index, edge_attr, ew1, eb1, ew2, eb2, nw1, nb1, nw2, nb2):
    raise NotImplementedError("write your pallas kernel here")


# speedup vs baseline: 3.0341x; 3.0341x over previous
import functools

import jax
import jax.numpy as jnp
from jax import lax
from jax.experimental import pallas as pl
from jax.experimental.pallas import tpu as pltpu
from jax.experimental.pallas import tpu_sc as plsc

N = 10000
E = 320000
DN = 128
DE = 16

NC = 2
NS = 16
NW = NC * NS
EPW = E // NW
CH = 80
NCHUNK = EPW // CH
NPAD = 10240
RPS = NPAD // NS

_MESH = plsc.VectorSubcoreMesh(core_axis_name="c", subcore_axis_name="s")


@functools.partial(
    pl.kernel,
    out_type=jax.ShapeDtypeStruct((E, DN), jnp.float32),
    mesh=_MESH,
    scratch_types=[
        pltpu.VMEM((CH,), jnp.int32),
        pltpu.VMEM((CH,), jnp.int32),
        pltpu.VMEM((CH, DN), jnp.float32),
        pltpu.VMEM((CH, DN), jnp.float32),
        pltpu.SemaphoreType.DMA,
        pltpu.SemaphoreType.DMA,
    ],
)
def _sc_gather(a_hbm, b_hbm, src_hbm, dst_hbm, s_hbm,
               idxa, idxb, rowsa, rowsb, sem1, sem2):
    wid = lax.axis_index("s") * NC + lax.axis_index("c")

    def chunk(k, carry):
        base = wid * EPW + k * CH
        pltpu.sync_copy(src_hbm.at[pl.ds(base, CH)], idxa)
        pltpu.sync_copy(dst_hbm.at[pl.ds(base, CH)], idxb)
        ca = pltpu.async_copy(a_hbm.at[idxa], rowsa, sem1)
        cb = pltpu.async_copy(b_hbm.at[idxb], rowsb, sem2)
        ca.wait()
        cb.wait()

        def add_row(i, c2):
            for d in range(DN // 16):
                sl = pl.ds(d * 16, 16)
                rowsa[i, sl] = rowsa[i, sl] + rowsb[i, sl]
            return c2

        lax.fori_loop(0, CH, add_row, 0)
        pltpu.sync_copy(rowsa, s_hbm.at[pl.ds(base, CH)])
        return carry

    lax.fori_loop(0, NCHUNK, chunk, 0)


@functools.partial(
    pl.kernel,
    out_type=jax.ShapeDtypeStruct((NC, NPAD, DN), jnp.float32),
    mesh=_MESH,
    scratch_types=[
        pltpu.VMEM((CH,), jnp.int32),
        pltpu.VMEM((CH, DN), jnp.float32),
        pltpu.VMEM_SHARED((NPAD, DN), jnp.float32),
    ],
)
def _sc_scatter(eo_hbm, src_hbm, zeros_hbm, out_hbm, idx, rows, agg_sh):
    c = lax.axis_index("c")
    s = lax.axis_index("s")
    wid = s * NC + c
    pltpu.sync_copy(zeros_hbm.at[pl.ds(s * RPS, RPS)],
                    agg_sh.at[pl.ds(s * RPS, RPS)])
    plsc.subcore_barrier()

    def chunk(k, carry):
        base = wid * EPW + k * CH
        pltpu.sync_copy(src_hbm.at[pl.ds(base, CH)], idx)
        pltpu.sync_copy(eo_hbm.at[pl.ds(base, CH)], rows)
        pltpu.sync_copy(rows, agg_sh.at[idx], add=True)
        return carry

    lax.fori_loop(0, NCHUNK, chunk, 0)
    plsc.subcore_barrier()
    pltpu.sync_copy(agg_sh.at[pl.ds(s * RPS, RPS)],
                    out_hbm.at[c, pl.ds(s * RPS, RPS)])


def _k1_body(nf_ref, wa_ref, wb_ref, wn_ref, a_ref, b_ref, nf1_ref):
    x = nf_ref[...]
    a_ref[...] = jnp.dot(x, wa_ref[...], preferred_element_type=jnp.float32)
    b_ref[...] = jnp.dot(x, wb_ref[...], preferred_element_type=jnp.float32)
    nf1_ref[...] = jnp.dot(x, wn_ref[...], preferred_element_type=jnp.float32)


def _k3_body(s_ref, ea_ref, w1c_ref, eb1_ref, ew2_ref, eb2_ref, eo_ref):
    h = s_ref[...] + jnp.dot(ea_ref[...], w1c_ref[...],
                             preferred_element_type=jnp.float32) + eb1_ref[...]
    h = jnp.maximum(h, 0.0)
    eo_ref[...] = jnp.dot(h, ew2_ref[...],
                          preferred_element_type=jnp.float32) + eb2_ref[...]


def _k5_body(p_ref, nf1_ref, nw1b_ref, nb1_ref, nw2_ref, nb2_ref, out_ref):
    agg = p_ref[0] + p_ref[1]
    nh = nf1_ref[...] + jnp.dot(agg, nw1b_ref[...],
                                preferred_element_type=jnp.float32) + nb1_ref[...]
    nh = jnp.maximum(nh, 0.0)
    out_ref[...] = jnp.dot(nh, nw2_ref[...],
                           preferred_element_type=jnp.float32) + nb2_ref[...]


_RN = 2000
_BE = 2000


def _full(shape):
    return pl.BlockSpec(shape, lambda i: tuple(0 for _ in shape))


def kernel(node_feats, edge_index, edge_attr, ew1, eb1, ew2, eb2,
           nw1, nb1, nw2, nb2):
    f32 = jnp.float32
    src = edge_index[0].astype(jnp.int32)
    dst = edge_index[1].astype(jnp.int32)
    w1a = ew1[:DN]
    w1b = ew1[DN:2 * DN]
    w1c = ew1[2 * DN:]
    nw1a = nw1[:DN]
    nw1b = nw1[DN:]
    eb1r = eb1.reshape(1, -1)
    eb2r = eb2.reshape(1, -1)
    nb1r = nb1.reshape(1, -1)
    nb2r = nb2.reshape(1, -1)

    a_tab, b_tab, nf1 = pl.pallas_call(
        _k1_body,
        grid=(N // _RN,),
        in_specs=[pl.BlockSpec((_RN, DN), lambda i: (i, 0)),
                  _full((DN, DN)), _full((DN, DN)), _full((DN, DN))],
        out_specs=[pl.BlockSpec((_RN, DN), lambda i: (i, 0))] * 3,
        out_shape=[jax.ShapeDtypeStruct((N, DN), f32)] * 3,
    )(node_feats, w1a, w1b, nw1a)

    s_sum = _sc_gather(a_tab, b_tab, src, dst)

    edge_out = pl.pallas_call(
        _k3_body,
        grid=(E // _BE,),
        in_specs=[pl.BlockSpec((_BE, DN), lambda i: (i, 0)),
                  pl.BlockSpec((_BE, DE), lambda i: (i, 0)),
                  _full((DE, DN)), _full((1, DN)),
                  _full((DN, DN)), _full((1, DN))],
        out_specs=pl.BlockSpec((_BE, DN), lambda i: (i, 0)),
        out_shape=jax.ShapeDtypeStruct((E, DN), f32),
    )(s_sum, edge_attr, w1c, eb1r, ew2, eb2r)

    zeros = jnp.zeros((NPAD, DN), f32)
    partials = _sc_scatter(edge_out, src, zeros)[:, :N]

    node_out = pl.pallas_call(
        _k5_body,
        grid=(N // _RN,),
        in_specs=[pl.BlockSpec((NC, _RN, DN), lambda i: (0, i, 0)),
                  pl.BlockSpec((_RN, DN), lambda i: (i, 0)),
                  _full((DN, DN)), _full((1, DN)),
                  _full((DN, DN)), _full((1, DN))],
        out_specs=pl.BlockSpec((_RN, DN), lambda i: (i, 0)),
        out_shape=jax.ShapeDtypeStruct((N, DN), f32),
    )(partials, nf1, nw1b, nb1r, nw2, nb2r)

    return (node_out, edge_out)


# speedup vs baseline: 4.7523x; 1.5663x over previous
import functools

import jax
import jax.numpy as jnp
from jax import lax
from jax.experimental import pallas as pl
from jax.experimental.pallas import tpu as pltpu
from jax.experimental.pallas import tpu_sc as plsc

N = 10000
E = 320000
DN = 128
DE = 16

NC = 2
NS = 16
NW = NC * NS
EPW = E // NW
CH = 80
NCH = EPW // CH
NPAD = 10240
RPS = NPAD // NS


_MESH = plsc.VectorSubcoreMesh(core_axis_name="c", subcore_axis_name="s")


def _vadd_rows(dst, a, b, slot):
    @plsc.parallel_loop(0, CH, unroll=2)
    def _(i):
        for d in range(DN // 16):
            sl = pl.ds(d * 16, 16)
            dst[slot, i, sl] = a[slot, i, sl] + b[slot, i, sl]


@functools.partial(
    pl.kernel,
    out_type=jax.ShapeDtypeStruct((E, DN), jnp.float32),
    mesh=_MESH,
    scratch_types=[
        pltpu.VMEM((NCH, CH), jnp.int32),
        pltpu.VMEM((NCH, CH), jnp.int32),
        pltpu.VMEM((2, CH, DN), jnp.float32),
        pltpu.VMEM((2, CH, DN), jnp.float32),
        pltpu.VMEM((2, CH, DN), jnp.float32),
        pltpu.SemaphoreType.DMA,
        pltpu.SemaphoreType.DMA,
        pltpu.SemaphoreType.DMA,
        pltpu.SemaphoreType.DMA,
        pltpu.SemaphoreType.DMA,
        pltpu.SemaphoreType.DMA,
    ],
)
def _sc_gather(a_hbm, b_hbm, src_hbm, dst_hbm, s_hbm,
               idxs, idxd, ra, rb, ro, ga0, ga1, gb0, gb1, st0, st1):
    wid = lax.axis_index("s") * NC + lax.axis_index("c")
    ga = [ga0, ga1]
    gb = [gb0, gb1]
    st = [st0, st1]

    pltpu.sync_copy(src_hbm.at[wid], idxs)
    pltpu.sync_copy(dst_hbm.at[wid], idxd)

    def fire(k, slot):
        pltpu.async_copy(a_hbm.at[idxs.at[k]], ra.at[slot], ga[slot])
        pltpu.async_copy(b_hbm.at[idxd.at[k]], rb.at[slot], gb[slot])

    def wait_gather(slot):
        pltpu.make_async_copy(a_hbm.at[idxs.at[0]], ra.at[slot], ga[slot]).wait()
        pltpu.make_async_copy(b_hbm.at[idxd.at[0]], rb.at[slot], gb[slot]).wait()

    def store(k, slot):
        pltpu.async_copy(ro.at[slot], s_hbm.at[pl.ds((wid * NCH + k) * CH, CH)],
                         st[slot])

    def wait_store(slot):
        pltpu.make_async_copy(ro.at[slot], s_hbm.at[pl.ds(0, CH)],
                              st[slot]).wait()

    fire(0, 0)
    fire(1, 1)
    wait_gather(0)
    _vadd_rows(ro, ra, rb, 0)
    store(0, 0)
    fire(2, 0)
    wait_gather(1)
    _vadd_rows(ro, ra, rb, 1)
    store(1, 1)

    def body(p, carry):
        k0 = 2 * p
        fire(k0 + 1, 1)
        wait_gather(0)
        wait_store(0)
        _vadd_rows(ro, ra, rb, 0)
        store(k0, 0)
        fire(k0 + 2, 0)
        wait_gather(1)
        wait_store(1)
        _vadd_rows(ro, ra, rb, 1)
        store(k0 + 1, 1)
        return carry

    lax.fori_loop(1, (NCH - 1) // 2, body, 0)

    wait_gather(0)
    wait_store(0)
    _vadd_rows(ro, ra, rb, 0)
    store(NCH - 1, 0)
    wait_store(0)
    wait_store(1)


@functools.partial(
    pl.kernel,
    out_type=jax.ShapeDtypeStruct((NC, NPAD, DN), jnp.float32),
    mesh=_MESH,
    scratch_types=[
        pltpu.VMEM((NCH, CH), jnp.int32),
        pltpu.VMEM((2, CH, DN), jnp.float32),
        pltpu.VMEM_SHARED((NPAD, DN), jnp.float32),
        pltpu.SemaphoreType.DMA,
        pltpu.SemaphoreType.DMA,
        pltpu.SemaphoreType.DMA,
        pltpu.SemaphoreType.DMA,
    ],
)
def _sc_scatter(eo_hbm, src_hbm, zeros_hbm, out_hbm,
                idxs, rows, agg_sh, ld0, ld1, sc0, sc1):
    c = lax.axis_index("c")
    s = lax.axis_index("s")
    wid = s * NC + c
    ld = [ld0, ld1]
    sc = [sc0, sc1]

    pltpu.sync_copy(src_hbm.at[wid], idxs)
    pltpu.sync_copy(zeros_hbm.at[pl.ds(s * RPS, RPS)],
                    agg_sh.at[pl.ds(s * RPS, RPS)])
    plsc.subcore_barrier()

    def load(m, slot):
        pltpu.async_copy(eo_hbm.at[pl.ds(wid * EPW + m * CH, CH)],
                         rows.at[slot], ld[slot])

    def wait_load(slot):
        pltpu.make_async_copy(eo_hbm.at[pl.ds(0, CH)], rows.at[slot],
                              ld[slot]).wait()

    def scat(m, slot):
        pltpu.async_copy(rows.at[slot], agg_sh.at[idxs.at[m]],
                         sc[slot], add=True)

    def wait_scat(slot):
        pltpu.make_async_copy(rows.at[slot], agg_sh.at[idxs.at[0]],
                              sc[slot]).wait()

    load(0, 0)
    load(1, 1)
    wait_load(0)
    scat(0, 0)
    wait_scat(0)
    load(2, 0)
    wait_load(1)
    scat(1, 1)

    def body(p, carry):
        m0 = 2 * p
        wait_scat(1)
        load(m0 + 1, 1)
        wait_load(0)
        scat(m0, 0)
        wait_scat(0)
        load(m0 + 2, 0)
        wait_load(1)
        scat(m0 + 1, 1)
        return carry

    lax.fori_loop(1, (NCH - 1) // 2, body, 0)

    wait_scat(1)
    wait_load(0)
    scat(NCH - 1, 0)
    wait_scat(0)

    plsc.subcore_barrier()
    pltpu.sync_copy(agg_sh.at[pl.ds(s * RPS, RPS)],
                    out_hbm.at[c, pl.ds(s * RPS, RPS)])


def _k1_body(nf_ref, wa_ref, wb_ref, wn_ref, a_ref, b_ref, nf1_ref):
    x = nf_ref[...]
    a_ref[...] = jnp.dot(x, wa_ref[...], preferred_element_type=jnp.float32)
    b_ref[...] = jnp.dot(x, wb_ref[...], preferred_element_type=jnp.float32)
    nf1_ref[...] = jnp.dot(x, wn_ref[...], preferred_element_type=jnp.float32)


def _k3_body(s_ref, ea_ref, w1c_ref, eb1_ref, ew2_ref, eb2_ref, eo_ref):
    h = s_ref[...] + jnp.dot(ea_ref[...], w1c_ref[...],
                             preferred_element_type=jnp.float32) + eb1_ref[...]
    h = jnp.maximum(h, 0.0)
    eo_ref[...] = jnp.dot(h, ew2_ref[...],
                          preferred_element_type=jnp.float32) + eb2_ref[...]


def _k5_body(p_ref, nf1_ref, nw1b_ref, nb1_ref, nw2_ref, nb2_ref, out_ref):
    agg = p_ref[0] + p_ref[1]
    nh = nf1_ref[...] + jnp.dot(agg, nw1b_ref[...],
                                preferred_element_type=jnp.float32) + nb1_ref[...]
    nh = jnp.maximum(nh, 0.0)
    out_ref[...] = jnp.dot(nh, nw2_ref[...],
                           preferred_element_type=jnp.float32) + nb2_ref[...]


_RN = 2000
_BE = 2000


def _full(shape):
    return pl.BlockSpec(shape, lambda i: tuple(0 for _ in shape))


def kernel(node_feats, edge_index, edge_attr, ew1, eb1, ew2, eb2,
           nw1, nb1, nw2, nb2):
    f32 = jnp.float32
    src = edge_index[0].astype(jnp.int32)
    dst = edge_index[1].astype(jnp.int32)
    src3 = src.reshape(NW, NCH, CH)
    dst3 = dst.reshape(NW, NCH, CH)
    w1a = ew1[:DN]
    w1b = ew1[DN:2 * DN]
    w1c = ew1[2 * DN:]
    nw1a = nw1[:DN]
    nw1b = nw1[DN:]
    eb1r = eb1.reshape(1, -1)
    eb2r = eb2.reshape(1, -1)
    nb1r = nb1.reshape(1, -1)
    nb2r = nb2.reshape(1, -1)

    a_tab, b_tab, nf1 = pl.pallas_call(
        _k1_body,
        grid=(N // _RN,),
        in_specs=[pl.BlockSpec((_RN, DN), lambda i: (i, 0)),
                  _full((DN, DN)), _full((DN, DN)), _full((DN, DN))],
        out_specs=[pl.BlockSpec((_RN, DN), lambda i: (i, 0))] * 3,
        out_shape=[jax.ShapeDtypeStruct((N, DN), f32)] * 3,
    )(node_feats, w1a, w1b, nw1a)

    s_sum = _sc_gather(a_tab, b_tab, src3, dst3)

    edge_out = pl.pallas_call(
        _k3_body,
        grid=(E // _BE,),
        in_specs=[pl.BlockSpec((_BE, DN), lambda i: (i, 0)),
                  pl.BlockSpec((_BE, DE), lambda i: (i, 0)),
                  _full((DE, DN)), _full((1, DN)),
                  _full((DN, DN)), _full((1, DN))],
        out_specs=pl.BlockSpec((_BE, DN), lambda i: (i, 0)),
        out_shape=jax.ShapeDtypeStruct((E, DN), f32),
    )(s_sum, edge_attr, w1c, eb1r, ew2, eb2r)

    zeros = jnp.zeros((NPAD, DN), f32)
    partials = _sc_scatter(edge_out, src3, zeros)[:, :N]

    node_out = pl.pallas_call(
        _k5_body,
        grid=(N // _RN,),
        in_specs=[pl.BlockSpec((NC, _RN, DN), lambda i: (0, i, 0)),
                  pl.BlockSpec((_RN, DN), lambda i: (i, 0)),
                  _full((DN, DN)), _full((1, DN)),
                  _full((DN, DN)), _full((1, DN))],
        out_specs=pl.BlockSpec((_RN, DN), lambda i: (i, 0)),
        out_shape=jax.ShapeDtypeStruct((N, DN), f32),
    )(partials, nf1, nw1b, nb1r, nw2, nb2r)

    return (node_out, edge_out)


# speedup vs baseline: 5.8560x; 1.2322x over previous
import functools

import jax
import jax.numpy as jnp
from jax import lax
from jax.experimental import pallas as pl
from jax.experimental.pallas import tpu as pltpu
from jax.experimental.pallas import tpu_sc as plsc

N = 10000
E = 320000
DN = 128
DE = 16

NC = 2
NS = 16
NW = NC * NS
EPW = E // NW
CH = 80
NCH = EPW // CH
NPAD = 10240
RPS = NPAD // NS

NCH0 = 63
NCH1 = NCH - NCH0
E0 = NW * NCH0 * CH
E1 = E - E0

_MESH = plsc.VectorSubcoreMesh(core_axis_name="c", subcore_axis_name="s")


def _vadd_rows(dst, a, b, slot):
    @plsc.parallel_loop(0, CH, unroll=2)
    def _(i):
        for d in range(DN // 16):
            sl = pl.ds(d * 16, 16)
            dst[slot, i, sl] = a[slot, i, sl] + b[slot, i, sl]


def _make_sc_gather(nch):
    e_g = NW * nch * CH

    @functools.partial(
        pl.kernel,
        out_type=jax.ShapeDtypeStruct((e_g, DN), jnp.float32),
        mesh=_MESH,
        scratch_types=[
            pltpu.VMEM((nch, CH), jnp.int32),
            pltpu.VMEM((nch, CH), jnp.int32),
            pltpu.VMEM((2, CH, DN), jnp.float32),
            pltpu.VMEM((2, CH, DN), jnp.float32),
            pltpu.VMEM((2, CH, DN), jnp.float32),
            pltpu.SemaphoreType.DMA,
            pltpu.SemaphoreType.DMA,
            pltpu.SemaphoreType.DMA,
            pltpu.SemaphoreType.DMA,
            pltpu.SemaphoreType.DMA,
            pltpu.SemaphoreType.DMA,
        ],
    )
    def sc_gather(a_hbm, b_hbm, src_hbm, dst_hbm, s_hbm,
                  idxs, idxd, ra, rb, ro, ga0, ga1, gb0, gb1, st0, st1):
        wid = lax.axis_index("s") * NC + lax.axis_index("c")
        ga = [ga0, ga1]
        gb = [gb0, gb1]
        st = [st0, st1]

        pltpu.sync_copy(src_hbm.at[wid], idxs)
        pltpu.sync_copy(dst_hbm.at[wid], idxd)

        def fire(k, slot):
            pltpu.async_copy(a_hbm.at[idxs.at[k]], ra.at[slot], ga[slot])
            pltpu.async_copy(b_hbm.at[idxd.at[k]], rb.at[slot], gb[slot])

        def wait_gather(slot):
            pltpu.make_async_copy(a_hbm.at[idxs.at[0]], ra.at[slot],
                                  ga[slot]).wait()
            pltpu.make_async_copy(b_hbm.at[idxd.at[0]], rb.at[slot],
                                  gb[slot]).wait()

        def store(k, slot):
            pltpu.async_copy(ro.at[slot],
                             s_hbm.at[pl.ds((wid * nch + k) * CH, CH)],
                             st[slot])

        def wait_store(slot):
            pltpu.make_async_copy(ro.at[slot], s_hbm.at[pl.ds(0, CH)],
                                  st[slot]).wait()

        fire(0, 0)
        fire(1, 1)
        wait_gather(0)
        _vadd_rows(ro, ra, rb, 0)
        store(0, 0)
        fire(2, 0)
        wait_gather(1)
        _vadd_rows(ro, ra, rb, 1)
        store(1, 1)

        even = nch % 2 == 0

        def body(p, carry):
            k0 = 2 * p
            fire(k0 + 1, 1)
            wait_gather(0)
            wait_store(0)
            _vadd_rows(ro, ra, rb, 0)
            store(k0, 0)
            if even:
                @pl.when(k0 + 2 < nch)
                def _():
                    fire(k0 + 2, 0)
            else:
                fire(k0 + 2, 0)
            wait_gather(1)
            wait_store(1)
            _vadd_rows(ro, ra, rb, 1)
            store(k0 + 1, 1)
            return carry

        lax.fori_loop(1, nch // 2, body, 0)

        if not even:
            wait_gather(0)
            wait_store(0)
            _vadd_rows(ro, ra, rb, 0)
            store(nch - 1, 0)
        wait_store(0)
        wait_store(1)

    return sc_gather


_sc_gather0 = _make_sc_gather(NCH0)
_sc_gather1 = _make_sc_gather(NCH1)


@functools.partial(
    pl.kernel,
    out_type=jax.ShapeDtypeStruct((NC, NPAD, DN), jnp.float32),
    mesh=_MESH,
    scratch_types=[
        pltpu.VMEM((NCH, CH), jnp.int32),
        pltpu.VMEM((2, CH, DN), jnp.float32),
        pltpu.VMEM_SHARED((NPAD, DN), jnp.float32),
        pltpu.SemaphoreType.DMA,
        pltpu.SemaphoreType.DMA,
        pltpu.SemaphoreType.DMA,
        pltpu.SemaphoreType.DMA,
    ],
)
def _sc_scatter(eo_hbm, src_hbm, zeros_hbm, out_hbm,
                idxs, rows, agg_sh, ld0, ld1, sc0, sc1):
    c = lax.axis_index("c")
    s = lax.axis_index("s")
    wid = s * NC + c
    ld = [ld0, ld1]
    sc = [sc0, sc1]

    pltpu.sync_copy(src_hbm.at[wid], idxs)
    pltpu.sync_copy(zeros_hbm.at[pl.ds(s * RPS, RPS)],
                    agg_sh.at[pl.ds(s * RPS, RPS)])
    plsc.subcore_barrier()

    def load(m, slot):
        pltpu.async_copy(eo_hbm.at[pl.ds(wid * EPW + m * CH, CH)],
                         rows.at[slot], ld[slot])

    def wait_load(slot):
        pltpu.make_async_copy(eo_hbm.at[pl.ds(0, CH)], rows.at[slot],
                              ld[slot]).wait()

    def scat(m, slot):
        pltpu.async_copy(rows.at[slot], agg_sh.at[idxs.at[m]],
                         sc[slot], add=True)

    def wait_scat(slot):
        pltpu.make_async_copy(rows.at[slot], agg_sh.at[idxs.at[0]],
                              sc[slot]).wait()

    load(0, 0)
    load(1, 1)
    wait_load(0)
    scat(0, 0)
    wait_scat(0)
    load(2, 0)
    wait_load(1)
    scat(1, 1)

    def body(p, carry):
        m0 = 2 * p
        wait_scat(1)
        load(m0 + 1, 1)
        wait_load(0)
        scat(m0, 0)
        wait_scat(0)
        load(m0 + 2, 0)
        wait_load(1)
        scat(m0 + 1, 1)
        return carry

    lax.fori_loop(1, (NCH - 1) // 2, body, 0)

    wait_scat(1)
    wait_load(0)
    scat(NCH - 1, 0)
    wait_scat(0)

    plsc.subcore_barrier()
    pltpu.sync_copy(agg_sh.at[pl.ds(s * RPS, RPS)],
                    out_hbm.at[c, pl.ds(s * RPS, RPS)])


def _k1_body(nf_ref, wa_ref, wb_ref, wn_ref, a_ref, b_ref, nf1_ref):
    x = nf_ref[...]
    a_ref[...] = jnp.dot(x, wa_ref[...], preferred_element_type=jnp.float32)
    b_ref[...] = jnp.dot(x, wb_ref[...], preferred_element_type=jnp.float32)
    nf1_ref[...] = jnp.dot(x, wn_ref[...], preferred_element_type=jnp.float32)


def _k3_compute(s_ref, eat_ref, w1c_ref, eb1_ref, ew2_ref, eb2_ref, eo_ref):
    c = lax.dot_general(eat_ref[...], w1c_ref[...],
                        dimension_numbers=(((0,), (0,)), ((), ())),
                        preferred_element_type=jnp.float32)
    h = jnp.maximum(s_ref[...] + c + eb1_ref[...], 0.0)
    eo_ref[...] = jnp.dot(h, ew2_ref[...],
                          preferred_element_type=jnp.float32) + eb2_ref[...]


def _k3_body0(s_ref, eat_ref, w1c_ref, eb1_ref, ew2_ref, eb2_ref, eo_ref):
    _k3_compute(s_ref, eat_ref, w1c_ref, eb1_ref, ew2_ref, eb2_ref, eo_ref)


def _k3_body1(prev_ref, s_ref, eat_ref, w1c_ref, eb1_ref, ew2_ref, eb2_ref,
              eo_ref):
    del prev_ref
    _k3_compute(s_ref, eat_ref, w1c_ref, eb1_ref, ew2_ref, eb2_ref, eo_ref)


def _k5_body(p_ref, nf1_ref, nw1b_ref, nb1_ref, nw2_ref, nb2_ref, out_ref):
    agg = p_ref[0] + p_ref[1]
    nh = nf1_ref[...] + jnp.dot(agg, nw1b_ref[...],
                                preferred_element_type=jnp.float32) + nb1_ref[...]
    nh = jnp.maximum(nh, 0.0)
    out_ref[...] = jnp.dot(nh, nw2_ref[...],
                           preferred_element_type=jnp.float32) + nb2_ref[...]


_RN = 2000
_BE = 2560


def _full(shape):
    return pl.BlockSpec(shape, lambda i: tuple(0 for _ in shape))


def kernel(node_feats, edge_index, edge_attr, ew1, eb1, ew2, eb2,
           nw1, nb1, nw2, nb2):
    f32 = jnp.float32
    src = edge_index[0].astype(jnp.int32)
    dst = edge_index[1].astype(jnp.int32)
    src3 = src.reshape(NW, NCH, CH)
    dst3 = dst.reshape(NW, NCH, CH)
    src3_0 = src[:E0].reshape(NW, NCH0, CH)
    dst3_0 = dst[:E0].reshape(NW, NCH0, CH)
    src3_1 = src[E0:].reshape(NW, NCH1, CH)
    dst3_1 = dst[E0:].reshape(NW, NCH1, CH)
    eat = edge_attr.T
    w1a = ew1[:DN]
    w1b = ew1[DN:2 * DN]
    w1c = ew1[2 * DN:]
    nw1a = nw1[:DN]
    nw1b = nw1[DN:]
    eb1r = eb1.reshape(1, -1)
    eb2r = eb2.reshape(1, -1)
    nb1r = nb1.reshape(1, -1)
    nb2r = nb2.reshape(1, -1)

    a_tab, b_tab, nf1 = pl.pallas_call(
        _k1_body,
        grid=(N // _RN,),
        in_specs=[pl.BlockSpec((_RN, DN), lambda i: (i, 0)),
                  _full((DN, DN)), _full((DN, DN)), _full((DN, DN))],
        out_specs=[pl.BlockSpec((_RN, DN), lambda i: (i, 0))] * 3,
        out_shape=[jax.ShapeDtypeStruct((N, DN), f32)] * 3,
    )(node_feats, w1a, w1b, nw1a)

    s_g0 = _sc_gather0(a_tab, b_tab, src3_0, dst3_0)
    s_g1 = _sc_gather1(a_tab, b_tab, src3_1, dst3_1)

    nblk0 = E0 // _BE
    nblk1 = E1 // _BE
    eo_g0 = pl.pallas_call(
        _k3_body0,
        grid=(nblk0,),
        in_specs=[pl.BlockSpec((_BE, DN), lambda i: (i, 0)),
                  pl.BlockSpec((DE, _BE), lambda i: (0, i)),
                  _full((DE, DN)), _full((1, DN)),
                  _full((DN, DN)), _full((1, DN))],
        out_specs=pl.BlockSpec((_BE, DN), lambda i: (i, 0)),
        out_shape=jax.ShapeDtypeStruct((E, DN), f32),
    )(s_g0, eat[:, :E0], w1c, eb1r, ew2, eb2r)
    edge_out = pl.pallas_call(
        _k3_body1,
        grid=(nblk1,),
        in_specs=[pl.BlockSpec(memory_space=pltpu.MemorySpace.HBM),
                  pl.BlockSpec((_BE, DN), lambda i: (i, 0)),
                  pl.BlockSpec((DE, _BE), lambda i: (0, i)),
                  _full((DE, DN)), _full((1, DN)),
                  _full((DN, DN)), _full((1, DN))],
        out_specs=pl.BlockSpec((_BE, DN), lambda i: (i + nblk0, 0)),
        input_output_aliases={0: 0},
        out_shape=jax.ShapeDtypeStruct((E, DN), f32),
    )(eo_g0, s_g1, eat[:, E0:], w1c, eb1r, ew2, eb2r)

    zeros = jnp.zeros((NPAD, DN), f32)
    partials = _sc_scatter(edge_out, src3, zeros)[:, :N]

    node_out = pl.pallas_call(
        _k5_body,
        grid=(N // _RN,),
        in_specs=[pl.BlockSpec((NC, _RN, DN), lambda i: (0, i, 0)),
                  pl.BlockSpec((_RN, DN), lambda i: (i, 0)),
                  _full((DN, DN)), _full((1, DN)),
                  _full((DN, DN)), _full((1, DN))],
        out_specs=pl.BlockSpec((_RN, DN), lambda i: (i, 0)),
        out_shape=jax.ShapeDtypeStruct((N, DN), f32),
    )(partials, nf1, nw1b, nb1r, nw2, nb2r)

    return (node_out, edge_out)


# speedup vs baseline: 5.9516x; 1.0163x over previous
import functools

import jax
import jax.numpy as jnp
from jax import lax
from jax.experimental import pallas as pl
from jax.experimental.pallas import tpu as pltpu
from jax.experimental.pallas import tpu_sc as plsc

N = 10000
E = 320000
DN = 128
DE = 16

NC = 2
NS = 16
NW = NC * NS
EPW = E // NW
CH = 80
NCH = EPW // CH
NPAD = 10240
RPS = NPAD // NS

GRP = (32, 31, 31, 31)
EG = tuple(NW * c * CH for c in GRP)

_MESH = plsc.VectorSubcoreMesh(core_axis_name="c", subcore_axis_name="s")


def _vadd_rows(dst, a, b, slot):
    @plsc.parallel_loop(0, CH, unroll=2)
    def _(i):
        for d in range(DN // 16):
            sl = pl.ds(d * 16, 16)
            dst[slot, i, sl] = a[slot, i, sl] + b[slot, i, sl]


def _make_sc_gather(nch):
    e_g = NW * nch * CH

    @functools.partial(
        pl.kernel,
        out_type=jax.ShapeDtypeStruct((e_g, DN), jnp.float32),
        mesh=_MESH,
        scratch_types=[
            pltpu.VMEM((nch, CH), jnp.int32),
            pltpu.VMEM((nch, CH), jnp.int32),
            pltpu.VMEM((2, CH, DN), jnp.float32),
            pltpu.VMEM((2, CH, DN), jnp.float32),
            pltpu.VMEM((2, CH, DN), jnp.float32),
            pltpu.SemaphoreType.DMA,
            pltpu.SemaphoreType.DMA,
            pltpu.SemaphoreType.DMA,
            pltpu.SemaphoreType.DMA,
            pltpu.SemaphoreType.DMA,
            pltpu.SemaphoreType.DMA,
        ],
    )
    def sc_gather(a_hbm, b_hbm, src_hbm, dst_hbm, s_hbm,
                  idxs, idxd, ra, rb, ro, ga0, ga1, gb0, gb1, st0, st1):
        wid = lax.axis_index("s") * NC + lax.axis_index("c")
        ga = [ga0, ga1]
        gb = [gb0, gb1]
        st = [st0, st1]

        pltpu.sync_copy(src_hbm.at[wid], idxs)
        pltpu.sync_copy(dst_hbm.at[wid], idxd)

        def fire(k, slot):
            pltpu.async_copy(a_hbm.at[idxs.at[k]], ra.at[slot], ga[slot])
            pltpu.async_copy(b_hbm.at[idxd.at[k]], rb.at[slot], gb[slot])

        def wait_gather(slot):
            pltpu.make_async_copy(a_hbm.at[idxs.at[0]], ra.at[slot],
                                  ga[slot]).wait()
            pltpu.make_async_copy(b_hbm.at[idxd.at[0]], rb.at[slot],
                                  gb[slot]).wait()

        def store(k, slot):
            pltpu.async_copy(ro.at[slot],
                             s_hbm.at[pl.ds((wid * nch + k) * CH, CH)],
                             st[slot])

        def wait_store(slot):
            pltpu.make_async_copy(ro.at[slot], s_hbm.at[pl.ds(0, CH)],
                                  st[slot]).wait()

        fire(0, 0)
        fire(1, 1)
        wait_gather(0)
        _vadd_rows(ro, ra, rb, 0)
        store(0, 0)
        fire(2, 0)
        wait_gather(1)
        _vadd_rows(ro, ra, rb, 1)
        store(1, 1)

        even = nch % 2 == 0

        def body(p, carry):
            k0 = 2 * p
            fire(k0 + 1, 1)
            wait_gather(0)
            wait_store(0)
            _vadd_rows(ro, ra, rb, 0)
            store(k0, 0)
            if even:
                @pl.when(k0 + 2 < nch)
                def _():
                    fire(k0 + 2, 0)
            else:
                fire(k0 + 2, 0)
            wait_gather(1)
            wait_store(1)
            _vadd_rows(ro, ra, rb, 1)
            store(k0 + 1, 1)
            return carry

        lax.fori_loop(1, nch // 2, body, 0)

        if not even:
            wait_gather(0)
            wait_store(0)
            _vadd_rows(ro, ra, rb, 0)
            store(nch - 1, 0)
        wait_store(0)
        wait_store(1)

    return sc_gather


_sc_gathers = [_make_sc_gather(c) for c in GRP]


@functools.partial(
    pl.kernel,
    out_type=jax.ShapeDtypeStruct((NC, NPAD, DN), jnp.float32),
    mesh=_MESH,
    scratch_types=[
        pltpu.VMEM((NCH, CH), jnp.int32),
        pltpu.VMEM((2, CH, DN), jnp.float32),
        pltpu.VMEM_SHARED((NPAD, DN), jnp.float32),
        pltpu.SemaphoreType.DMA,
        pltpu.SemaphoreType.DMA,
        pltpu.SemaphoreType.DMA,
        pltpu.SemaphoreType.DMA,
    ],
)
def _sc_scatter(eo_hbm, src_hbm, zeros_hbm, out_hbm,
                idxs, rows, agg_sh, ld0, ld1, sc0, sc1):
    c = lax.axis_index("c")
    s = lax.axis_index("s")
    wid = s * NC + c
    ld = [ld0, ld1]
    sc = [sc0, sc1]

    pltpu.sync_copy(src_hbm.at[wid], idxs)
    pltpu.sync_copy(zeros_hbm.at[pl.ds(s * RPS, RPS)],
                    agg_sh.at[pl.ds(s * RPS, RPS)])
    plsc.subcore_barrier()

    def load(m, slot):
        pltpu.async_copy(eo_hbm.at[pl.ds(wid * EPW + m * CH, CH)],
                         rows.at[slot], ld[slot])

    def wait_load(slot):
        pltpu.make_async_copy(eo_hbm.at[pl.ds(0, CH)], rows.at[slot],
                              ld[slot]).wait()

    def scat(m, slot):
        pltpu.async_copy(rows.at[slot], agg_sh.at[idxs.at[m]],
                         sc[slot], add=True)

    def wait_scat(slot):
        pltpu.make_async_copy(rows.at[slot], agg_sh.at[idxs.at[0]],
                              sc[slot]).wait()

    load(0, 0)
    load(1, 1)
    wait_load(0)
    scat(0, 0)
    wait_scat(0)
    load(2, 0)
    wait_load(1)
    scat(1, 1)

    def body(p, carry):
        m0 = 2 * p
        wait_scat(1)
        load(m0 + 1, 1)
        wait_load(0)
        scat(m0, 0)
        wait_scat(0)
        load(m0 + 2, 0)
        wait_load(1)
        scat(m0 + 1, 1)
        return carry

    lax.fori_loop(1, (NCH - 1) // 2, body, 0)

    wait_scat(1)
    wait_load(0)
    scat(NCH - 1, 0)
    wait_scat(0)

    plsc.subcore_barrier()
    pltpu.sync_copy(agg_sh.at[pl.ds(s * RPS, RPS)],
                    out_hbm.at[c, pl.ds(s * RPS, RPS)])


def _k1_body(nf_ref, wa_ref, wb_ref, wn_ref, a_ref, b_ref, nf1_ref):
    x = nf_ref[...]
    a_ref[...] = jnp.dot(x, wa_ref[...], preferred_element_type=jnp.float32)
    b_ref[...] = jnp.dot(x, wb_ref[...], preferred_element_type=jnp.float32)
    nf1_ref[...] = jnp.dot(x, wn_ref[...], preferred_element_type=jnp.float32)


def _k3_compute(s_ref, eat_ref, w1c_ref, eb1_ref, ew2_ref, eb2_ref, eo_ref):
    c = lax.dot_general(eat_ref[...], w1c_ref[...],
                        dimension_numbers=(((0,), (0,)), ((), ())),
                        preferred_element_type=jnp.float32)
    h = jnp.maximum(s_ref[...] + c + eb1_ref[...], 0.0)
    eo_ref[...] = jnp.dot(h, ew2_ref[...],
                          preferred_element_type=jnp.float32) + eb2_ref[...]


def _k3_body0(s_ref, eat_ref, w1c_ref, eb1_ref, ew2_ref, eb2_ref, eo_ref):
    _k3_compute(s_ref, eat_ref, w1c_ref, eb1_ref, ew2_ref, eb2_ref, eo_ref)


def _k3_body1(prev_ref, s_ref, eat_ref, w1c_ref, eb1_ref, ew2_ref, eb2_ref,
              eo_ref):
    del prev_ref
    _k3_compute(s_ref, eat_ref, w1c_ref, eb1_ref, ew2_ref, eb2_ref, eo_ref)


def _k5_body(p_ref, nf1_ref, nw1b_ref, nb1_ref, nw2_ref, nb2_ref, out_ref):
    agg = p_ref[0] + p_ref[1]
    nh = nf1_ref[...] + jnp.dot(agg, nw1b_ref[...],
                                preferred_element_type=jnp.float32) + nb1_ref[...]
    nh = jnp.maximum(nh, 0.0)
    out_ref[...] = jnp.dot(nh, nw2_ref[...],
                           preferred_element_type=jnp.float32) + nb2_ref[...]


_RN = 2000
_BE = 2560


def _full(shape):
    return pl.BlockSpec(shape, lambda i: tuple(0 for _ in shape))


def kernel(node_feats, edge_index, edge_attr, ew1, eb1, ew2, eb2,
           nw1, nb1, nw2, nb2):
    f32 = jnp.float32
    src = edge_index[0].astype(jnp.int32)
    dst = edge_index[1].astype(jnp.int32)
    src3 = src.reshape(NW, NCH, CH)
    dst3 = dst.reshape(NW, NCH, CH)
    eat = edge_attr.T
    w1a = ew1[:DN]
    w1b = ew1[DN:2 * DN]
    w1c = ew1[2 * DN:]
    nw1a = nw1[:DN]
    nw1b = nw1[DN:]
    eb1r = eb1.reshape(1, -1)
    eb2r = eb2.reshape(1, -1)
    nb1r = nb1.reshape(1, -1)
    nb2r = nb2.reshape(1, -1)

    a_tab, b_tab, nf1 = pl.pallas_call(
        _k1_body,
        grid=(N // _RN,),
        in_specs=[pl.BlockSpec((_RN, DN), lambda i: (i, 0)),
                  _full((DN, DN)), _full((DN, DN)), _full((DN, DN))],
        out_specs=[pl.BlockSpec((_RN, DN), lambda i: (i, 0))] * 3,
        out_shape=[jax.ShapeDtypeStruct((N, DN), f32)] * 3,
    )(node_feats, w1a, w1b, nw1a)

    bounds = []
    acc = 0
    for eg in EG:
        bounds.append((acc, acc + eg))
        acc += eg
    s_gs = []
    for gi, (lo, hi) in enumerate(bounds):
        c = GRP[gi]
        src3_g = src[lo:hi].reshape(NW, c, CH)
        dst3_g = dst[lo:hi].reshape(NW, c, CH)
        s_gs.append(_sc_gathers[gi](a_tab, b_tab, src3_g, dst3_g))

    eo = None
    blk_off = 0
    for gi, (lo, hi) in enumerate(bounds):
        nblk = EG[gi] // _BE
        off = blk_off
        out_spec = pl.BlockSpec((_BE, DN), lambda i, off=off: (i + off, 0))
        in_specs = [pl.BlockSpec((_BE, DN), lambda i: (i, 0)),
                    pl.BlockSpec((DE, _BE), lambda i: (0, i)),
                    _full((DE, DN)), _full((1, DN)),
                    _full((DN, DN)), _full((1, DN))]
        if gi == 0:
            eo = pl.pallas_call(
                _k3_body0,
                grid=(nblk,),
                in_specs=in_specs,
                out_specs=out_spec,
                out_shape=jax.ShapeDtypeStruct((E, DN), f32),
            )(s_gs[gi], eat[:, lo:hi], w1c, eb1r, ew2, eb2r)
        else:
            eo = pl.pallas_call(
                _k3_body1,
                grid=(nblk,),
                in_specs=[pl.BlockSpec(memory_space=pltpu.MemorySpace.HBM)]
                         + in_specs,
                out_specs=out_spec,
                input_output_aliases={0: 0},
                out_shape=jax.ShapeDtypeStruct((E, DN), f32),
            )(eo, s_gs[gi], eat[:, lo:hi], w1c, eb1r, ew2, eb2r)
        blk_off += nblk
    edge_out = eo

    zeros = jnp.zeros((NPAD, DN), f32)
    partials = _sc_scatter(edge_out, src3, zeros)[:, :N]

    node_out = pl.pallas_call(
        _k5_body,
        grid=(N // _RN,),
        in_specs=[pl.BlockSpec((NC, _RN, DN), lambda i: (0, i, 0)),
                  pl.BlockSpec((_RN, DN), lambda i: (i, 0)),
                  _full((DN, DN)), _full((1, DN)),
                  _full((DN, DN)), _full((1, DN))],
        out_specs=pl.BlockSpec((_RN, DN), lambda i: (i, 0)),
        out_shape=jax.ShapeDtypeStruct((N, DN), f32),
    )(partials, nf1, nw1b, nb1r, nw2, nb2r)

    return (node_out, edge_out)


# speedup vs baseline: 6.0447x; 1.0157x over previous
import functools

import jax
import jax.numpy as jnp
from jax import lax
from jax.experimental import pallas as pl
from jax.experimental.pallas import tpu as pltpu
from jax.experimental.pallas import tpu_sc as plsc

N = 10000
E = 320000
DN = 128
DE = 16

NC = 2
NS = 16
NW = NC * NS
EPW = E // NW
CH = 80
NCH = EPW // CH
NPAD = 10240
RPS = NPAD // NS

GRP = (32, 31, 31, 31)
EG = tuple(NW * c * CH for c in GRP)

_MESH = plsc.VectorSubcoreMesh(core_axis_name="c", subcore_axis_name="s")


def _vadd_rows(dst, a, b, slot):
    @plsc.parallel_loop(0, CH, unroll=2)
    def _(i):
        for d in range(DN // 16):
            sl = pl.ds(d * 16, 16)
            dst[slot, i, sl] = a[slot, i, sl] + b[slot, i, sl]


def _make_sc_gather(nch):
    e_g = NW * nch * CH

    @functools.partial(
        pl.kernel,
        out_type=jax.ShapeDtypeStruct((e_g, DN), jnp.float32),
        mesh=_MESH,
        scratch_types=[
            pltpu.VMEM((nch, CH), jnp.int32),
            pltpu.VMEM((nch, CH), jnp.int32),
            pltpu.VMEM((2, CH, DN), jnp.float32),
            pltpu.VMEM((2, CH, DN), jnp.float32),
            pltpu.VMEM((2, CH, DN), jnp.float32),
            pltpu.SemaphoreType.DMA,
            pltpu.SemaphoreType.DMA,
            pltpu.SemaphoreType.DMA,
            pltpu.SemaphoreType.DMA,
            pltpu.SemaphoreType.DMA,
            pltpu.SemaphoreType.DMA,
        ],
    )
    def sc_gather(a_hbm, b_hbm, src_hbm, dst_hbm, s_hbm,
                  idxs, idxd, ra, rb, ro, ga0, ga1, gb0, gb1, st0, st1):
        wid = lax.axis_index("s") * NC + lax.axis_index("c")
        ga = [ga0, ga1]
        gb = [gb0, gb1]
        st = [st0, st1]

        pltpu.sync_copy(src_hbm.at[wid], idxs)
        pltpu.sync_copy(dst_hbm.at[wid], idxd)

        def fire(k, slot):
            pltpu.async_copy(a_hbm.at[idxs.at[k]], ra.at[slot], ga[slot])
            pltpu.async_copy(b_hbm.at[idxd.at[k]], rb.at[slot], gb[slot])

        def wait_gather(slot):
            pltpu.make_async_copy(a_hbm.at[idxs.at[0]], ra.at[slot],
                                  ga[slot]).wait()
            pltpu.make_async_copy(b_hbm.at[idxd.at[0]], rb.at[slot],
                                  gb[slot]).wait()

        def store(k, slot):
            pltpu.async_copy(ro.at[slot],
                             s_hbm.at[pl.ds((wid * nch + k) * CH, CH)],
                             st[slot])

        def wait_store(slot):
            pltpu.make_async_copy(ro.at[slot], s_hbm.at[pl.ds(0, CH)],
                                  st[slot]).wait()

        fire(0, 0)
        fire(1, 1)
        wait_gather(0)
        _vadd_rows(ro, ra, rb, 0)
        store(0, 0)
        fire(2, 0)
        wait_gather(1)
        _vadd_rows(ro, ra, rb, 1)
        store(1, 1)

        even = nch % 2 == 0

        def body(p, carry):
            k0 = 2 * p
            fire(k0 + 1, 1)
            wait_gather(0)
            wait_store(0)
            _vadd_rows(ro, ra, rb, 0)
            store(k0, 0)
            if even:
                @pl.when(k0 + 2 < nch)
                def _():
                    fire(k0 + 2, 0)
            else:
                fire(k0 + 2, 0)
            wait_gather(1)
            wait_store(1)
            _vadd_rows(ro, ra, rb, 1)
            store(k0 + 1, 1)
            return carry

        lax.fori_loop(1, nch // 2, body, 0)

        if not even:
            wait_gather(0)
            wait_store(0)
            _vadd_rows(ro, ra, rb, 0)
            store(nch - 1, 0)
        wait_store(0)
        wait_store(1)

    return sc_gather


_sc_gathers = [_make_sc_gather(c) for c in GRP]


@functools.partial(
    pl.kernel,
    out_type=jax.ShapeDtypeStruct((NC, NPAD, DN), jnp.float32),
    mesh=_MESH,
    scratch_types=[
        pltpu.VMEM((NCH, CH), jnp.int32),
        pltpu.VMEM((2, CH, DN), jnp.float32),
        pltpu.VMEM_SHARED((NPAD, DN), jnp.float32),
        pltpu.SemaphoreType.DMA,
        pltpu.SemaphoreType.DMA,
        pltpu.SemaphoreType.DMA,
        pltpu.SemaphoreType.DMA,
    ],
)
def _sc_scatter(eo_hbm, src_hbm, zeros_hbm, out_hbm,
                idxs, rows, agg_sh, ld0, ld1, sc0, sc1):
    c = lax.axis_index("c")
    s = lax.axis_index("s")
    wid = s * NC + c
    ld = [ld0, ld1]
    sc = [sc0, sc1]

    pltpu.sync_copy(src_hbm.at[wid], idxs)
    pltpu.sync_copy(zeros_hbm.at[pl.ds(s * RPS, RPS)],
                    agg_sh.at[pl.ds(s * RPS, RPS)])
    plsc.subcore_barrier()

    def load(m, slot):
        pltpu.async_copy(eo_hbm.at[pl.ds(wid * EPW + m * CH, CH)],
                         rows.at[slot], ld[slot])

    def wait_load(slot):
        pltpu.make_async_copy(eo_hbm.at[pl.ds(0, CH)], rows.at[slot],
                              ld[slot]).wait()

    def scat(m, slot):
        pltpu.async_copy(rows.at[slot], agg_sh.at[idxs.at[m]],
                         sc[slot], add=True)

    def wait_scat(slot):
        pltpu.make_async_copy(rows.at[slot], agg_sh.at[idxs.at[0]],
                              sc[slot]).wait()

    load(0, 0)
    load(1, 1)
    wait_load(0)
    scat(0, 0)
    wait_scat(0)
    load(2, 0)
    wait_load(1)
    scat(1, 1)

    def body(p, carry):
        m0 = 2 * p
        wait_scat(1)
        load(m0 + 1, 1)
        wait_load(0)
        scat(m0, 0)
        wait_scat(0)
        load(m0 + 2, 0)
        wait_load(1)
        scat(m0 + 1, 1)
        return carry

    lax.fori_loop(1, (NCH - 1) // 2, body, 0)

    wait_scat(1)
    wait_load(0)
    scat(NCH - 1, 0)
    wait_scat(0)

    plsc.subcore_barrier()
    pltpu.sync_copy(agg_sh.at[pl.ds(s * RPS, RPS)],
                    out_hbm.at[c, pl.ds(s * RPS, RPS)])


def _k1_body(nf_ref, wa_ref, wb_ref, wn_ref, a_ref, b_ref, nf1_ref):
    x = nf_ref[...]
    a_ref[...] = jnp.dot(x, wa_ref[...], preferred_element_type=jnp.float32)
    b_ref[...] = jnp.dot(x, wb_ref[...], preferred_element_type=jnp.float32)
    nf1_ref[...] = jnp.dot(x, wn_ref[...], preferred_element_type=jnp.float32)


def _k3_compute(s_ref, eat_ref, w1c_ref, eb1_ref, ew2_ref, eb2_ref, eo_ref):
    c = lax.dot_general(eat_ref[...], w1c_ref[...],
                        dimension_numbers=(((0,), (0,)), ((), ())),
                        preferred_element_type=jnp.float32)
    h = jnp.maximum(s_ref[...] + c + eb1_ref[...], 0.0)
    eo_ref[...] = jnp.dot(h, ew2_ref[...],
                          preferred_element_type=jnp.float32) + eb2_ref[...]


def _k3_body0(s_ref, eat_ref, w1c_ref, eb1_ref, ew2_ref, eb2_ref, eo_ref):
    _k3_compute(s_ref, eat_ref, w1c_ref, eb1_ref, ew2_ref, eb2_ref, eo_ref)


def _k3_body1(prev_ref, s_ref, eat_ref, w1c_ref, eb1_ref, ew2_ref, eb2_ref,
              eo_ref):
    del prev_ref
    _k3_compute(s_ref, eat_ref, w1c_ref, eb1_ref, ew2_ref, eb2_ref, eo_ref)


def _k5_body(p_ref, nf1_ref, nw1b_ref, nb1_ref, nw2_ref, nb2_ref, out_ref):
    agg = p_ref[0] + p_ref[1]
    nh = nf1_ref[...] + jnp.dot(agg, nw1b_ref[...],
                                preferred_element_type=jnp.float32) + nb1_ref[...]
    nh = jnp.maximum(nh, 0.0)
    out_ref[...] = jnp.dot(nh, nw2_ref[...],
                           preferred_element_type=jnp.float32) + nb2_ref[...]


_RN = 2000
_BE = 2560


def _full(shape):
    return pl.BlockSpec(shape, lambda i: tuple(0 for _ in shape))


def kernel(node_feats, edge_index, edge_attr, ew1, eb1, ew2, eb2,
           nw1, nb1, nw2, nb2):
    f32 = jnp.float32
    src = edge_index[0].astype(jnp.int32)
    dst = edge_index[1].astype(jnp.int32)
    src3 = src.reshape(NW, NCH, CH)
    dst3 = dst.reshape(NW, NCH, CH)
    eat = edge_attr.T
    w1a = ew1[:DN]
    w1b = ew1[DN:2 * DN]
    w1c = ew1[2 * DN:]
    nw1a = nw1[:DN]
    nw1b = nw1[DN:]
    eb1r = eb1.reshape(1, -1)
    eb2r = eb2.reshape(1, -1)
    nb1r = nb1.reshape(1, -1)
    nb2r = nb2.reshape(1, -1)

    a_tab, b_tab, nf1 = pl.pallas_call(
        _k1_body,
        grid=(N // _RN,),
        in_specs=[pl.BlockSpec((_RN, DN), lambda i: (i, 0)),
                  _full((DN, DN)), _full((DN, DN)), _full((DN, DN))],
        out_specs=[pl.BlockSpec((_RN, DN), lambda i: (i, 0))] * 3,
        out_shape=[jax.ShapeDtypeStruct((N, DN), f32)] * 3,
    )(node_feats, w1a, w1b, nw1a)

    bounds = []
    acc = 0
    for eg in EG:
        bounds.append((acc, acc + eg))
        acc += eg
    s_gs = []
    for gi, (lo, hi) in enumerate(bounds):
        c = GRP[gi]
        src3_g = src[lo:hi].reshape(NW, c, CH)
        dst3_g = dst[lo:hi].reshape(NW, c, CH)
        s_gs.append(_sc_gathers[gi](a_tab, b_tab, src3_g, dst3_g))

    eo = None
    blk_off = 0
    for gi, (lo, hi) in enumerate(bounds):
        nblk = EG[gi] // _BE
        off = blk_off
        out_spec = pl.BlockSpec((_BE, DN), lambda i, off=off: (i + off, 0))
        in_specs = [pl.BlockSpec((_BE, DN), lambda i: (i, 0)),
                    pl.BlockSpec((DE, _BE), lambda i: (0, i)),
                    _full((DE, DN)), _full((1, DN)),
                    _full((DN, DN)), _full((1, DN))]
        if gi == 0:
            eo = pl.pallas_call(
                _k3_body0,
                grid=(nblk,),
                in_specs=in_specs,
                out_specs=out_spec,
                out_shape=jax.ShapeDtypeStruct((E, DN), f32),
            )(s_gs[gi], eat[:, lo:hi], w1c, eb1r, ew2, eb2r)
        else:
            eo = pl.pallas_call(
                _k3_body1,
                grid=(nblk,),
                in_specs=[pl.BlockSpec(memory_space=pltpu.MemorySpace.HBM)]
                         + in_specs,
                out_specs=out_spec,
                input_output_aliases={0: 0},
                out_shape=jax.ShapeDtypeStruct((E, DN), f32),
            )(eo, s_gs[gi], eat[:, lo:hi], w1c, eb1r, ew2, eb2r)
        blk_off += nblk
    edge_out = eo

    zeros = jnp.zeros((NPAD, DN), f32)
    partials = _sc_scatter(edge_out, src3, zeros)

    node_out = pl.pallas_call(
        _k5_body,
        grid=(N // _RN,),
        in_specs=[pl.BlockSpec((NC, _RN, DN), lambda i: (0, i, 0)),
                  pl.BlockSpec((_RN, DN), lambda i: (i, 0)),
                  _full((DN, DN)), _full((1, DN)),
                  _full((DN, DN)), _full((1, DN))],
        out_specs=pl.BlockSpec((_RN, DN), lambda i: (i, 0)),
        out_shape=jax.ShapeDtypeStruct((N, DN), f32),
    )(partials, nf1, nw1b, nb1r, nw2, nb2r)

    return (node_out, edge_out)


# speedup vs baseline: 6.1164x; 1.0119x over previous
import functools

import jax
import jax.numpy as jnp
from jax import lax
from jax.experimental import pallas as pl
from jax.experimental.pallas import tpu as pltpu
from jax.experimental.pallas import tpu_sc as plsc

N = 10000
E = 320000
DN = 128
DE = 16

NC = 2
NS = 16
NW = NC * NS
EPW = E // NW
CH = 80
NCH = EPW // CH
NPAD = 10240
RPS = NPAD // NS

GRP = (32, 31, 31, 31)
EG = tuple(NW * c * CH for c in GRP)

_MESH = plsc.VectorSubcoreMesh(core_axis_name="c", subcore_axis_name="s")


def _vadd_rows(dst, a, b, slot):
    @plsc.parallel_loop(0, CH, unroll=2)
    def _(i):
        for d in range(DN // 16):
            sl = pl.ds(d * 16, 16)
            dst[slot, i, sl] = a[slot, i, sl] + b[slot, i, sl]


def _make_sc_gather(nch):
    e_g = NW * nch * CH

    @functools.partial(
        pl.kernel,
        out_type=jax.ShapeDtypeStruct((e_g, DN), jnp.float32),
        mesh=_MESH,
        scratch_types=[
            pltpu.VMEM((nch, CH), jnp.int32),
            pltpu.VMEM((nch, CH), jnp.int32),
            pltpu.VMEM((2, CH, DN), jnp.float32),
            pltpu.VMEM((2, CH, DN), jnp.float32),
            pltpu.VMEM((2, CH, DN), jnp.float32),
            pltpu.SemaphoreType.DMA,
            pltpu.SemaphoreType.DMA,
            pltpu.SemaphoreType.DMA,
            pltpu.SemaphoreType.DMA,
            pltpu.SemaphoreType.DMA,
            pltpu.SemaphoreType.DMA,
        ],
    )
    def sc_gather(a_hbm, b_hbm, src_hbm, dst_hbm, s_hbm,
                  idxs, idxd, ra, rb, ro, ga0, ga1, gb0, gb1, st0, st1):
        wid = lax.axis_index("s") * NC + lax.axis_index("c")
        ga = [ga0, ga1]
        gb = [gb0, gb1]
        st = [st0, st1]

        pltpu.sync_copy(src_hbm.at[wid], idxs)
        pltpu.sync_copy(dst_hbm.at[wid], idxd)

        def fire(k, slot):
            pltpu.async_copy(a_hbm.at[idxs.at[k]], ra.at[slot], ga[slot])
            pltpu.async_copy(b_hbm.at[idxd.at[k]], rb.at[slot], gb[slot])

        def wait_gather(slot):
            pltpu.make_async_copy(a_hbm.at[idxs.at[0]], ra.at[slot],
                                  ga[slot]).wait()
            pltpu.make_async_copy(b_hbm.at[idxd.at[0]], rb.at[slot],
                                  gb[slot]).wait()

        def store(k, slot):
            pltpu.async_copy(ro.at[slot],
                             s_hbm.at[pl.ds((wid * nch + k) * CH, CH)],
                             st[slot])

        def wait_store(slot):
            pltpu.make_async_copy(ro.at[slot], s_hbm.at[pl.ds(0, CH)],
                                  st[slot]).wait()

        fire(0, 0)
        fire(1, 1)
        wait_gather(0)
        _vadd_rows(ro, ra, rb, 0)
        store(0, 0)
        fire(2, 0)
        wait_gather(1)
        _vadd_rows(ro, ra, rb, 1)
        store(1, 1)

        even = nch % 2 == 0

        def body(p, carry):
            k0 = 2 * p
            fire(k0 + 1, 1)
            wait_gather(0)
            wait_store(0)
            _vadd_rows(ro, ra, rb, 0)
            store(k0, 0)
            if even:
                @pl.when(k0 + 2 < nch)
                def _():
                    fire(k0 + 2, 0)
            else:
                fire(k0 + 2, 0)
            wait_gather(1)
            wait_store(1)
            _vadd_rows(ro, ra, rb, 1)
            store(k0 + 1, 1)
            return carry

        lax.fori_loop(1, nch // 2, body, 0)

        if not even:
            wait_gather(0)
            wait_store(0)
            _vadd_rows(ro, ra, rb, 0)
            store(nch - 1, 0)
        wait_store(0)
        wait_store(1)

    return sc_gather


_sc_gathers = [_make_sc_gather(c) for c in GRP]


@functools.partial(
    pl.kernel,
    out_type=jax.ShapeDtypeStruct((NC, NPAD, DN), jnp.float32),
    mesh=_MESH,
    scratch_types=[
        pltpu.VMEM((NCH, CH), jnp.int32),
        pltpu.VMEM((2, CH, DN), jnp.float32),
        pltpu.VMEM_SHARED((NPAD, DN), jnp.float32),
        pltpu.SemaphoreType.DMA,
        pltpu.SemaphoreType.DMA,
        pltpu.SemaphoreType.DMA,
        pltpu.SemaphoreType.DMA,
    ],
)
def _sc_scatter(eo_hbm, src_hbm, out_hbm,
                idxs, rows, agg_sh, ld0, ld1, sc0, sc1):
    c = lax.axis_index("c")
    s = lax.axis_index("s")
    wid = s * NC + c
    ld = [ld0, ld1]
    sc = [sc0, sc1]

    pltpu.sync_copy(src_hbm.at[wid], idxs)
    zv = jnp.zeros((16,), jnp.float32)

    @plsc.parallel_loop(0, CH, unroll=2)
    def _(i):
        for d in range(DN // 16):
            rows[0, i, pl.ds(d * 16, 16)] = zv

    for j in range(RPS // CH):
        pltpu.sync_copy(rows.at[0], agg_sh.at[pl.ds(s * RPS + j * CH, CH)])
    plsc.subcore_barrier()

    def load(m, slot):
        pltpu.async_copy(eo_hbm.at[pl.ds(wid * EPW + m * CH, CH)],
                         rows.at[slot], ld[slot])

    def wait_load(slot):
        pltpu.make_async_copy(eo_hbm.at[pl.ds(0, CH)], rows.at[slot],
                              ld[slot]).wait()

    def scat(m, slot):
        pltpu.async_copy(rows.at[slot], agg_sh.at[idxs.at[m]],
                         sc[slot], add=True)

    def wait_scat(slot):
        pltpu.make_async_copy(rows.at[slot], agg_sh.at[idxs.at[0]],
                              sc[slot]).wait()

    load(0, 0)
    load(1, 1)
    wait_load(0)
    scat(0, 0)
    wait_scat(0)
    load(2, 0)
    wait_load(1)
    scat(1, 1)

    def body(p, carry):
        m0 = 2 * p
        wait_scat(1)
        load(m0 + 1, 1)
        wait_load(0)
        scat(m0, 0)
        wait_scat(0)
        load(m0 + 2, 0)
        wait_load(1)
        scat(m0 + 1, 1)
        return carry

    lax.fori_loop(1, (NCH - 1) // 2, body, 0)

    wait_scat(1)
    wait_load(0)
    scat(NCH - 1, 0)
    wait_scat(0)

    plsc.subcore_barrier()
    pltpu.sync_copy(agg_sh.at[pl.ds(s * RPS, RPS)],
                    out_hbm.at[c, pl.ds(s * RPS, RPS)])


def _k1_body(nf_ref, wa_ref, wb_ref, wn_ref, a_ref, b_ref, nf1_ref):
    x = nf_ref[...]
    a_ref[...] = jnp.dot(x, wa_ref[...], preferred_element_type=jnp.float32)
    b_ref[...] = jnp.dot(x, wb_ref[...], preferred_element_type=jnp.float32)
    nf1_ref[...] = jnp.dot(x, wn_ref[...], preferred_element_type=jnp.float32)


def _k3_compute(s_ref, eat_ref, w1c_ref, eb1_ref, ew2_ref, eb2_ref, eo_ref):
    c = lax.dot_general(eat_ref[...], w1c_ref[...],
                        dimension_numbers=(((0,), (0,)), ((), ())),
                        preferred_element_type=jnp.float32)
    h = jnp.maximum(s_ref[...] + c + eb1_ref[...], 0.0)
    eo_ref[...] = jnp.dot(h, ew2_ref[...],
                          preferred_element_type=jnp.float32) + eb2_ref[...]


def _k3_body0(s_ref, eat_ref, w1c_ref, eb1_ref, ew2_ref, eb2_ref, eo_ref):
    _k3_compute(s_ref, eat_ref, w1c_ref, eb1_ref, ew2_ref, eb2_ref, eo_ref)


def _k3_body1(prev_ref, s_ref, eat_ref, w1c_ref, eb1_ref, ew2_ref, eb2_ref,
              eo_ref):
    del prev_ref
    _k3_compute(s_ref, eat_ref, w1c_ref, eb1_ref, ew2_ref, eb2_ref, eo_ref)


def _k5_body(p_ref, nf1_ref, nw1b_ref, nb1_ref, nw2_ref, nb2_ref, out_ref):
    agg = p_ref[0] + p_ref[1]
    nh = nf1_ref[...] + jnp.dot(agg, nw1b_ref[...],
                                preferred_element_type=jnp.float32) + nb1_ref[...]
    nh = jnp.maximum(nh, 0.0)
    out_ref[...] = jnp.dot(nh, nw2_ref[...],
                           preferred_element_type=jnp.float32) + nb2_ref[...]


_RN = 2000
_BE = 2560


def _full(shape):
    return pl.BlockSpec(shape, lambda i: tuple(0 for _ in shape))


def kernel(node_feats, edge_index, edge_attr, ew1, eb1, ew2, eb2,
           nw1, nb1, nw2, nb2):
    f32 = jnp.float32
    src = edge_index[0].astype(jnp.int32)
    dst = edge_index[1].astype(jnp.int32)
    src3 = src.reshape(NW, NCH, CH)
    dst3 = dst.reshape(NW, NCH, CH)
    eat = edge_attr.T
    w1a = ew1[:DN]
    w1b = ew1[DN:2 * DN]
    w1c = ew1[2 * DN:]
    nw1a = nw1[:DN]
    nw1b = nw1[DN:]
    eb1r = eb1.reshape(1, -1)
    eb2r = eb2.reshape(1, -1)
    nb1r = nb1.reshape(1, -1)
    nb2r = nb2.reshape(1, -1)

    a_tab, b_tab, nf1 = pl.pallas_call(
        _k1_body,
        grid=(N // _RN,),
        in_specs=[pl.BlockSpec((_RN, DN), lambda i: (i, 0)),
                  _full((DN, DN)), _full((DN, DN)), _full((DN, DN))],
        out_specs=[pl.BlockSpec((_RN, DN), lambda i: (i, 0))] * 3,
        out_shape=[jax.ShapeDtypeStruct((N, DN), f32)] * 3,
    )(node_feats, w1a, w1b, nw1a)

    bounds = []
    acc = 0
    for eg in EG:
        bounds.append((acc, acc + eg))
        acc += eg
    s_gs = []
    for gi, (lo, hi) in enumerate(bounds):
        c = GRP[gi]
        src3_g = src[lo:hi].reshape(NW, c, CH)
        dst3_g = dst[lo:hi].reshape(NW, c, CH)
        s_gs.append(_sc_gathers[gi](a_tab, b_tab, src3_g, dst3_g))

    eo = None
    blk_off = 0
    for gi, (lo, hi) in enumerate(bounds):
        nblk = EG[gi] // _BE
        off = blk_off
        out_spec = pl.BlockSpec((_BE, DN), lambda i, off=off: (i + off, 0))
        in_specs = [pl.BlockSpec((_BE, DN), lambda i: (i, 0)),
                    pl.BlockSpec((DE, _BE), lambda i: (0, i)),
                    _full((DE, DN)), _full((1, DN)),
                    _full((DN, DN)), _full((1, DN))]
        if gi == 0:
            eo = pl.pallas_call(
                _k3_body0,
                grid=(nblk,),
                in_specs=in_specs,
                out_specs=out_spec,
                out_shape=jax.ShapeDtypeStruct((E, DN), f32),
            )(s_gs[gi], eat[:, lo:hi], w1c, eb1r, ew2, eb2r)
        else:
            eo = pl.pallas_call(
                _k3_body1,
                grid=(nblk,),
                in_specs=[pl.BlockSpec(memory_space=pltpu.MemorySpace.HBM)]
                         + in_specs,
                out_specs=out_spec,
                input_output_aliases={0: 0},
                out_shape=jax.ShapeDtypeStruct((E, DN), f32),
            )(eo, s_gs[gi], eat[:, lo:hi], w1c, eb1r, ew2, eb2r)
        blk_off += nblk
    edge_out = eo

    partials = _sc_scatter(edge_out, src3)

    node_out = pl.pallas_call(
        _k5_body,
        grid=(N // _RN,),
        in_specs=[pl.BlockSpec((NC, _RN, DN), lambda i: (0, i, 0)),
                  pl.BlockSpec((_RN, DN), lambda i: (i, 0)),
                  _full((DN, DN)), _full((1, DN)),
                  _full((DN, DN)), _full((1, DN))],
        out_specs=pl.BlockSpec((_RN, DN), lambda i: (i, 0)),
        out_shape=jax.ShapeDtypeStruct((N, DN), f32),
    )(partials, nf1, nw1b, nb1r, nw2, nb2r)

    return (node_out, edge_out)
